# baseline (speedup 1.0000x reference)
import functools

import jax
import jax.numpy as jnp
from jax import lax
from jax.experimental import pallas as pl
from jax.experimental.pallas import tpu as pltpu
from jax.experimental.pallas import tpu_sc as plsc

N = 10000
E = 320000
D_IN = 128
D_H = 128
D_OUT = 64
DC = 64

N1 = 10240
RPT = N1 // 16
NW = 32
NB = 80
B = 128
K = 4
EPW = NB * B
EPAD = NW * EPW

_mesh = plsc.VectorSubcoreMesh(core_axis_name="c", subcore_axis_name="s")
_sc_params = pltpu.CompilerParams(use_tc_tiling_on_sc=False)


def _wid():
    return lax.axis_index("c") * 16 + lax.axis_index("s")


@functools.partial(
    pl.kernel,
    out_type=(
        jax.ShapeDtypeStruct((2, N1, 16), jnp.float32),
        jax.ShapeDtypeStruct((2, N1, 16), jnp.float32),
    ),
    mesh=_mesh,
    compiler_params=_sc_params,
    scratch_types=[
        pltpu.VMEM((NB, B), jnp.int32),
        pltpu.VMEM((NB, B), jnp.int32),
        pltpu.VMEM((B, 16), jnp.float32),
        pltpu.VMEM_SHARED((N1, 16), jnp.float32),
        pltpu.VMEM_SHARED((N1, 16), jnp.float32),
        pltpu.SemaphoreType.DMA,
    ],
)
def _deg_kernel(src_hbm, dst_hbm, ones_hbm, z16_hbm,
                deg_out_hbm, deg_in_hbm,
                src_v, dst_v, ones_v, acc_o, acc_i, sem):
    c = lax.axis_index("c")
    s = lax.axis_index("s")
    w = _wid()
    r0 = s * RPT
    pltpu.sync_copy(z16_hbm, acc_o.at[pl.ds(r0, RPT)])
    pltpu.sync_copy(z16_hbm, acc_i.at[pl.ds(r0, RPT)])
    pltpu.sync_copy(src_hbm.at[w], src_v)
    pltpu.sync_copy(dst_hbm.at[w], dst_v)
    pltpu.sync_copy(ones_hbm, ones_v)
    plsc.subcore_barrier()

    def body(g, carry):
        for q in range(4):
            j = 4 * g + q
            pltpu.async_copy(ones_v, acc_o.at[src_v.at[j]], sem, add=True)
            pltpu.async_copy(ones_v, acc_i.at[dst_v.at[j]], sem, add=True)
        for q in range(8):
            pltpu.make_async_copy(ones_v, acc_i.at[dst_v.at[0]], sem).wait()
        return carry

    lax.fori_loop(0, NB // 4, body, 0)
    plsc.subcore_barrier()
    pltpu.sync_copy(acc_o.at[pl.ds(r0, RPT)], deg_out_hbm.at[c, pl.ds(r0, RPT)])
    pltpu.sync_copy(acc_i.at[pl.ds(r0, RPT)], deg_in_hbm.at[c, pl.ds(r0, RPT)])


@functools.partial(
    pl.kernel,
    out_type=jax.ShapeDtypeStruct((2, N1, DC), jnp.float32),
    mesh=_mesh,
    compiler_params=_sc_params,
    scratch_types=[
        pltpu.VMEM((NB, B), jnp.int32),
        pltpu.VMEM((NB, B), jnp.int32),
        [pltpu.VMEM((B, DC), jnp.float32)] * K,
        [pltpu.SemaphoreType.DMA] * K,
        [pltpu.SemaphoreType.DMA] * K,
        pltpu.VMEM_SHARED((N1, DC), jnp.float32),
    ],
)
def _agg_kernel(m_hbm, src_hbm, dst_hbm, z_hbm, out_hbm,
                src_v, dst_v, rows, gsem, ssem, acc):
    c = lax.axis_index("c")
    s = lax.axis_index("s")
    w = _wid()
    r0 = s * RPT
    pltpu.sync_copy(z_hbm, acc.at[pl.ds(r0, RPT)])
    pltpu.sync_copy(src_hbm.at[w], src_v)
    pltpu.sync_copy(dst_hbm.at[w], dst_v)
    plsc.subcore_barrier()

    def body(g, carry):
        for b in range(K):
            j = K * g + b

            @pl.when(g > 0)
            def _drain_scatter(b=b):
                pltpu.make_async_copy(rows[b], acc.at[dst_v.at[0]],
                                      ssem[b]).wait()

            pltpu.async_copy(m_hbm.at[src_v.at[j]], rows[b], gsem[b])
        for b in range(K):
            j = K * g + b
            pltpu.make_async_copy(m_hbm.at[src_v.at[0]], rows[b],
                                  gsem[b]).wait()
            pltpu.async_copy(rows[b], acc.at[dst_v.at[j]], ssem[b], add=True)
        return carry

    lax.fori_loop(0, NB // K, body, 0)
    for b in range(K):
        pltpu.make_async_copy(rows[b], acc.at[dst_v.at[0]], ssem[b]).wait()
    plsc.subcore_barrier()
    pltpu.sync_copy(acc.at[pl.ds(r0, RPT)], out_hbm.at[c, pl.ds(r0, RPT)])


def _norm_col(deg_ref):
    d = deg_ref[0] + deg_ref[1]
    return lax.rsqrt(jnp.maximum(d[:, 0:1], 1.0))


def _mm1_body(dego_ref, x_ref, w_ref, ha_ref, hb_ref):
    ns = _norm_col(dego_ref)
    h = jnp.dot(x_ref[...], w_ref[...], preferred_element_type=jnp.float32,
                precision=lax.Precision.HIGHEST)
    h = h * ns
    ha_ref[...] = h[:, :DC]
    hb_ref[...] = h[:, DC:]


def _mm2_body(pa_ref, pb_ref, dego_ref, degi_ref, w_ref, b_ref, h_ref):
    ns = _norm_col(dego_ref)
    nd = _norm_col(degi_ref)
    ta = jnp.maximum((pa_ref[0] + pa_ref[1]) * nd + b_ref[:, :DC], 0.0)
    tb = jnp.maximum((pb_ref[0] + pb_ref[1]) * nd + b_ref[:, DC:], 0.0)
    h = (jnp.dot(ta, w_ref[:DC], preferred_element_type=jnp.float32,
                 precision=lax.Precision.HIGHEST)
         + jnp.dot(tb, w_ref[DC:], preferred_element_type=jnp.float32,
                   precision=lax.Precision.HIGHEST))
    h_ref[...] = h * ns


def _fin_body(aggp_ref, degi_ref, b_ref, o_ref):
    nd = _norm_col(degi_ref)
    o_ref[...] = (aggp_ref[0] + aggp_ref[1]) * nd + b_ref[...]


_BLK = 640
_GRID = N1 // _BLK


def _deg_spec():
    return pl.BlockSpec((2, _BLK, 16), lambda i: (0, i, 0))


def _agg_spec():
    return pl.BlockSpec((2, _BLK, DC), lambda i: (0, i, 0))


def _tc_mm1(deg_out_p, x_pad, W1):
    return pl.pallas_call(
        _mm1_body,
        grid=(_GRID,),
        in_specs=[
            _deg_spec(),
            pl.BlockSpec((_BLK, D_IN), lambda i: (i, 0)),
            pl.BlockSpec((D_IN, D_H), lambda i: (0, 0)),
        ],
        out_specs=[
            pl.BlockSpec((_BLK, DC), lambda i: (i, 0)),
            pl.BlockSpec((_BLK, DC), lambda i: (i, 0)),
        ],
        out_shape=[
            jax.ShapeDtypeStruct((N1, DC), jnp.float32),
            jax.ShapeDtypeStruct((N1, DC), jnp.float32),
        ],
    )(deg_out_p, x_pad, W1)


def _tc_mm2(agg1a_p, agg1b_p, deg_out_p, deg_in_p, W2, b1):
    return pl.pallas_call(
        _mm2_body,
        grid=(_GRID,),
        in_specs=[
            _agg_spec(),
            _agg_spec(),
            _deg_spec(),
            _deg_spec(),
            pl.BlockSpec((D_H, D_OUT), lambda i: (0, 0)),
            pl.BlockSpec((1, D_H), lambda i: (0, 0)),
        ],
        out_specs=pl.BlockSpec((_BLK, D_OUT), lambda i: (i, 0)),
        out_shape=jax.ShapeDtypeStruct((N1, D_OUT), jnp.float32),
    )(agg1a_p, agg1b_p, deg_out_p, deg_in_p, W2, b1)


def _tc_fin(agg2_p, deg_in_p, b2):
    return pl.pallas_call(
        _fin_body,
        grid=(_GRID,),
        in_specs=[
            _agg_spec(),
            _deg_spec(),
            pl.BlockSpec((1, D_OUT), lambda i: (0, 0)),
        ],
        out_specs=pl.BlockSpec((_BLK, D_OUT), lambda i: (i, 0)),
        out_shape=jax.ShapeDtypeStruct((N1, D_OUT), jnp.float32),
    )(agg2_p, deg_in_p, b2)


def kernel(x, edge_index, W1, b1, W2, b2):
    src = edge_index[0]
    dst = edge_index[1]
    pad = jnp.full((EPAD - E,), N, dtype=jnp.int32)
    src_p = jnp.concatenate([src, pad]).reshape(NW, NB, B)
    dst_p = jnp.concatenate([dst, pad]).reshape(NW, NB, B)
    x_pad = jnp.zeros((N1, D_IN), jnp.float32).at[:N].set(x)

    ones16 = jnp.ones((B, 16), jnp.float32)
    z16 = jnp.zeros((RPT, 16), jnp.float32)
    z64 = jnp.zeros((RPT, DC), jnp.float32)

    deg_out_p, deg_in_p = _deg_kernel(src_p, dst_p, ones16, z16)
    h1a, h1b = _tc_mm1(deg_out_p, x_pad, W1)
    agg1a_p = _agg_kernel(h1a, src_p, dst_p, z64)
    agg1b_p = _agg_kernel(h1b, src_p, dst_p, z64)
    h2 = _tc_mm2(agg1a_p, agg1b_p, deg_out_p, deg_in_p, W2,
                 b1.reshape(1, D_H))
    agg2_p = _agg_kernel(h2, src_p, dst_p, z64)
    out = _tc_fin(agg2_p, deg_in_p, b2.reshape(1, D_OUT))
    return out[:N]


# speedup vs baseline: 1.0592x; 1.0592x over previous
import functools

import jax
import jax.numpy as jnp
from jax import lax
from jax.experimental import pallas as pl
from jax.experimental.pallas import tpu as pltpu
from jax.experimental.pallas import tpu_sc as plsc

N = 10000
E = 320000
D_IN = 128
D_H = 128
D_OUT = 64
DC = 64

N1 = 10240
RPT = N1 // 16
NW = 32
B = 128
K = 4
NBT = 2560
EPAD = NBT * B
FA, SA = 124, 36
FD, SD = 96, 64

_mesh = plsc.VectorSubcoreMesh(core_axis_name="c", subcore_axis_name="s")
_sc_params = pltpu.CompilerParams(use_tc_tiling_on_sc=False)


def _wid():
    return lax.axis_index("c") * 16 + lax.axis_index("s")


@functools.partial(
    pl.kernel,
    out_type=(
        jax.ShapeDtypeStruct((2, N1, 16), jnp.float32),
        jax.ShapeDtypeStruct((2, N1, 16), jnp.float32),
    ),
    mesh=_mesh,
    compiler_params=_sc_params,
    scratch_types=[
        pltpu.VMEM((FD, B), jnp.int32),
        pltpu.VMEM((FD, B), jnp.int32),
        pltpu.VMEM((B, 16), jnp.float32),
        pltpu.VMEM_SHARED((N1, 16), jnp.float32),
        pltpu.VMEM_SHARED((N1, 16), jnp.float32),
        pltpu.SemaphoreType.DMA,
    ],
)
def _deg_kernel(src_hbm, dst_hbm, ones_hbm, z16_hbm,
                deg_out_hbm, deg_in_hbm,
                src_v, dst_v, ones_v, acc_o, acc_i, sem):
    c = lax.axis_index("c")
    s = lax.axis_index("s")
    r0 = s * RPT
    pltpu.sync_copy(z16_hbm, acc_o.at[pl.ds(r0, RPT)])
    pltpu.sync_copy(z16_hbm, acc_i.at[pl.ds(r0, RPT)])
    pltpu.sync_copy(ones_hbm, ones_v)
    plsc.subcore_barrier()

    def run(count, start):
        pltpu.sync_copy(src_hbm.at[pl.ds(start, count)],
                        src_v.at[pl.ds(0, count)])
        pltpu.sync_copy(dst_hbm.at[pl.ds(start, count)],
                        dst_v.at[pl.ds(0, count)])

        def body(g, carry):
            for q in range(4):
                j = 4 * g + q
                pltpu.async_copy(ones_v, acc_o.at[src_v.at[j]], sem,
                                 add=True)
                pltpu.async_copy(ones_v, acc_i.at[dst_v.at[j]], sem,
                                 add=True)
            for q in range(8):
                pltpu.make_async_copy(ones_v, acc_i.at[dst_v.at[0]],
                                      sem).wait()
            return carry

        lax.fori_loop(0, count // 4, body, 0)

    @pl.when(c == 0)
    def _fast():
        run(FD, s * FD)

    @pl.when(c == 1)
    def _slow():
        run(SD, 16 * FD + s * SD)

    plsc.subcore_barrier()
    pltpu.sync_copy(acc_o.at[pl.ds(r0, RPT)], deg_out_hbm.at[c, pl.ds(r0, RPT)])
    pltpu.sync_copy(acc_i.at[pl.ds(r0, RPT)], deg_in_hbm.at[c, pl.ds(r0, RPT)])


@functools.partial(
    pl.kernel,
    out_type=jax.ShapeDtypeStruct((2, N1, DC), jnp.float32),
    mesh=_mesh,
    compiler_params=_sc_params,
    scratch_types=[
        pltpu.VMEM((FA, B), jnp.int32),
        pltpu.VMEM((FA, B), jnp.int32),
        [pltpu.VMEM((B, DC), jnp.float32)] * K,
        [pltpu.SemaphoreType.DMA] * K,
        [pltpu.SemaphoreType.DMA] * K,
        pltpu.VMEM_SHARED((N1, DC), jnp.float32),
    ],
)
def _agg_kernel(m_hbm, src_hbm, dst_hbm, z_hbm, out_hbm,
                src_v, dst_v, rows, gsem, ssem, acc):
    c = lax.axis_index("c")
    s = lax.axis_index("s")
    r0 = s * RPT
    pltpu.sync_copy(z_hbm, acc.at[pl.ds(r0, RPT)])
    plsc.subcore_barrier()

    def run(count, start):
        pltpu.sync_copy(src_hbm.at[pl.ds(start, count)],
                        src_v.at[pl.ds(0, count)])
        pltpu.sync_copy(dst_hbm.at[pl.ds(start, count)],
                        dst_v.at[pl.ds(0, count)])

        def body(g, carry):
            for b in range(K):
                j = K * g + b

                @pl.when(g > 0)
                def _drain_scatter(b=b):
                    pltpu.make_async_copy(rows[b], acc.at[dst_v.at[0]],
                                          ssem[b]).wait()

                pltpu.async_copy(m_hbm.at[src_v.at[j]], rows[b], gsem[b])
            for b in range(K):
                j = K * g + b
                pltpu.make_async_copy(m_hbm.at[src_v.at[0]], rows[b],
                                      gsem[b]).wait()
                pltpu.async_copy(rows[b], acc.at[dst_v.at[j]], ssem[b],
                                 add=True)
            return carry

        lax.fori_loop(0, count // K, body, 0)
        for b in range(K):
            pltpu.make_async_copy(rows[b], acc.at[dst_v.at[0]],
                                  ssem[b]).wait()

    @pl.when(c == 0)
    def _fast():
        run(FA, s * FA)

    @pl.when(c == 1)
    def _slow():
        run(SA, 16 * FA + s * SA)

    plsc.subcore_barrier()
    pltpu.sync_copy(acc.at[pl.ds(r0, RPT)], out_hbm.at[c, pl.ds(r0, RPT)])


def _norm_col(deg_ref):
    d = deg_ref[0] + deg_ref[1]
    return lax.rsqrt(jnp.maximum(d[:, 0:1], 1.0))


def _mm1_body(dego_ref, x_ref, w_ref, ha_ref, hb_ref):
    ns = _norm_col(dego_ref)
    h = jnp.dot(x_ref[...], w_ref[...], preferred_element_type=jnp.float32,
                precision=lax.Precision.HIGHEST)
    h = h * ns
    ha_ref[...] = h[:, :DC]
    hb_ref[...] = h[:, DC:]


def _mm2_body(pa_ref, pb_ref, dego_ref, degi_ref, w_ref, b_ref, h_ref):
    ns = _norm_col(dego_ref)
    nd = _norm_col(degi_ref)
    ta = jnp.maximum((pa_ref[0] + pa_ref[1]) * nd + b_ref[:, :DC], 0.0)
    tb = jnp.maximum((pb_ref[0] + pb_ref[1]) * nd + b_ref[:, DC:], 0.0)
    h = (jnp.dot(ta, w_ref[:DC], preferred_element_type=jnp.float32,
                 precision=lax.Precision.HIGHEST)
         + jnp.dot(tb, w_ref[DC:], preferred_element_type=jnp.float32,
                   precision=lax.Precision.HIGHEST))
    h_ref[...] = h * ns


def _fin_body(aggp_ref, degi_ref, b_ref, o_ref):
    nd = _norm_col(degi_ref)
    o_ref[...] = (aggp_ref[0] + aggp_ref[1]) * nd + b_ref[...]


_BLK = 640
_GRID = N1 // _BLK


def _deg_spec():
    return pl.BlockSpec((2, _BLK, 16), lambda i: (0, i, 0))


def _agg_spec():
    return pl.BlockSpec((2, _BLK, DC), lambda i: (0, i, 0))


def _tc_mm1(deg_out_p, x_pad, W1):
    return pl.pallas_call(
        _mm1_body,
        grid=(_GRID,),
        in_specs=[
            _deg_spec(),
            pl.BlockSpec((_BLK, D_IN), lambda i: (i, 0)),
            pl.BlockSpec((D_IN, D_H), lambda i: (0, 0)),
        ],
        out_specs=[
            pl.BlockSpec((_BLK, DC), lambda i: (i, 0)),
            pl.BlockSpec((_BLK, DC), lambda i: (i, 0)),
        ],
        out_shape=[
            jax.ShapeDtypeStruct((N1, DC), jnp.float32),
            jax.ShapeDtypeStruct((N1, DC), jnp.float32),
        ],
    )(deg_out_p, x_pad, W1)


def _tc_mm2(agg1a_p, agg1b_p, deg_out_p, deg_in_p, W2, b1):
    return pl.pallas_call(
        _mm2_body,
        grid=(_GRID,),
        in_specs=[
            _agg_spec(),
            _agg_spec(),
            _deg_spec(),
            _deg_spec(),
            pl.BlockSpec((D_H, D_OUT), lambda i: (0, 0)),
            pl.BlockSpec((1, D_H), lambda i: (0, 0)),
        ],
        out_specs=pl.BlockSpec((_BLK, D_OUT), lambda i: (i, 0)),
        out_shape=jax.ShapeDtypeStruct((N1, D_OUT), jnp.float32),
    )(agg1a_p, agg1b_p, deg_out_p, deg_in_p, W2, b1)


def _tc_fin(agg2_p, deg_in_p, b2):
    return pl.pallas_call(
        _fin_body,
        grid=(_GRID,),
        in_specs=[
            _agg_spec(),
            _deg_spec(),
            pl.BlockSpec((1, D_OUT), lambda i: (0, 0)),
        ],
        out_specs=pl.BlockSpec((_BLK, D_OUT), lambda i: (i, 0)),
        out_shape=jax.ShapeDtypeStruct((N1, D_OUT), jnp.float32),
    )(agg2_p, deg_in_p, b2)


def kernel(x, edge_index, W1, b1, W2, b2):
    src = edge_index[0]
    dst = edge_index[1]
    pad = jnp.full((EPAD - E,), N, dtype=jnp.int32)
    src_p = jnp.concatenate([src, pad]).reshape(NBT, B)
    dst_p = jnp.concatenate([dst, pad]).reshape(NBT, B)
    x_pad = jnp.zeros((N1, D_IN), jnp.float32).at[:N].set(x)

    ones16 = jnp.ones((B, 16), jnp.float32)
    z16 = jnp.zeros((RPT, 16), jnp.float32)
    z64 = jnp.zeros((RPT, DC), jnp.float32)

    deg_out_p, deg_in_p = _deg_kernel(src_p, dst_p, ones16, z16)
    h1a, h1b = _tc_mm1(deg_out_p, x_pad, W1)
    agg1a_p = _agg_kernel(h1a, src_p, dst_p, z64)
    agg1b_p = _agg_kernel(h1b, src_p, dst_p, z64)
    h2 = _tc_mm2(agg1a_p, agg1b_p, deg_out_p, deg_in_p, W2,
                 b1.reshape(1, D_H))
    agg2_p = _agg_kernel(h2, src_p, dst_p, z64)
    out = _tc_fin(agg2_p, deg_in_p, b2.reshape(1, D_OUT))
    return out[:N]


# speedup vs baseline: 1.7438x; 1.6463x over previous
import functools

import jax
import jax.numpy as jnp
from jax import lax
from jax.experimental import pallas as pl
from jax.experimental.pallas import tpu as pltpu
from jax.experimental.pallas import tpu_sc as plsc

N = 10000
E = 320000
D_IN = 128
D_H = 128
D_OUT = 64
DC = 64

N1 = 10240
RPT = N1 // 16
NW = 32
B = 128
K = 3
NB = 81
NBT = NW * NB
EPAD = NBT * B

_mesh = plsc.VectorSubcoreMesh(core_axis_name="c", subcore_axis_name="s")
_sc_params = pltpu.CompilerParams(use_tc_tiling_on_sc=False)


def _wid():
    return lax.axis_index("c") * 16 + lax.axis_index("s")


@functools.partial(
    pl.kernel,
    out_type=(
        jax.ShapeDtypeStruct((2, N1, 16), jnp.float32),
        jax.ShapeDtypeStruct((2, N1, 16), jnp.float32),
    ),
    mesh=_mesh,
    compiler_params=_sc_params,
    scratch_types=[
        pltpu.VMEM((NB, B), jnp.int32),
        pltpu.VMEM((NB, B), jnp.int32),
        pltpu.VMEM((B, 16), jnp.float32),
        pltpu.VMEM_SHARED((N1, 16), jnp.float32),
        pltpu.VMEM_SHARED((N1, 16), jnp.float32),
        pltpu.SemaphoreType.DMA,
    ],
)
def _deg_kernel(src_hbm, dst_hbm, ones_hbm, z16_hbm,
                deg_out_hbm, deg_in_hbm,
                src_v, dst_v, ones_v, acc_o, acc_i, sem):
    c = lax.axis_index("c")
    s = lax.axis_index("s")
    w = _wid()
    r0 = s * RPT
    pltpu.sync_copy(z16_hbm, acc_o.at[pl.ds(r0, RPT)])
    pltpu.sync_copy(z16_hbm, acc_i.at[pl.ds(r0, RPT)])
    pltpu.sync_copy(src_hbm.at[pl.ds(w * NB, NB)], src_v)
    pltpu.sync_copy(dst_hbm.at[pl.ds(w * NB, NB)], dst_v)
    pltpu.sync_copy(ones_hbm, ones_v)
    plsc.subcore_barrier()

    def body(g, carry):
        for q in range(K):
            j = K * g + q
            pltpu.async_copy(ones_v, acc_o.at[src_v.at[j]], sem, add=True)
            pltpu.async_copy(ones_v, acc_i.at[dst_v.at[j]], sem, add=True)
        for q in range(2 * K):
            pltpu.make_async_copy(ones_v, acc_i.at[dst_v.at[0]], sem).wait()
        return carry

    lax.fori_loop(0, NB // K, body, 0)
    plsc.subcore_barrier()
    pltpu.sync_copy(acc_o.at[pl.ds(r0, RPT)], deg_out_hbm.at[c, pl.ds(r0, RPT)])
    pltpu.sync_copy(acc_i.at[pl.ds(r0, RPT)], deg_in_hbm.at[c, pl.ds(r0, RPT)])


@functools.partial(
    pl.kernel,
    out_type=jax.ShapeDtypeStruct((2, N1, DC), jnp.float32),
    mesh=_mesh,
    compiler_params=_sc_params,
    scratch_types=[
        pltpu.VMEM((NB, B), jnp.int32),
        pltpu.VMEM((NB, B), jnp.int32),
        [pltpu.VMEM((B, DC), jnp.float32)] * K,
        [pltpu.SemaphoreType.DMA] * K,
        [pltpu.SemaphoreType.DMA] * K,
        pltpu.VMEM_SHARED((N1, DC), jnp.float32),
        pltpu.VMEM_SHARED((N1, DC), jnp.float32),
    ],
)
def _agg_kernel(m_hbm, src_hbm, dst_hbm, z_hbm, out_hbm,
                src_v, dst_v, rows, gsem, ssem, table, acc):
    c = lax.axis_index("c")
    s = lax.axis_index("s")
    w = _wid()
    r0 = s * RPT
    pltpu.sync_copy(m_hbm.at[pl.ds(r0, RPT)], table.at[pl.ds(r0, RPT)])
    pltpu.sync_copy(z_hbm, acc.at[pl.ds(r0, RPT)])
    pltpu.sync_copy(src_hbm.at[pl.ds(w * NB, NB)], src_v)
    pltpu.sync_copy(dst_hbm.at[pl.ds(w * NB, NB)], dst_v)
    plsc.subcore_barrier()

    def body(g, carry):
        for b in range(K):
            j = K * g + b

            @pl.when(g > 0)
            def _drain_scatter(b=b):
                pltpu.make_async_copy(rows[b], acc.at[dst_v.at[0]],
                                      ssem[b]).wait()

            pltpu.async_copy(table.at[src_v.at[j]], rows[b], gsem[b])
        for b in range(K):
            j = K * g + b
            pltpu.make_async_copy(table.at[src_v.at[0]], rows[b],
                                  gsem[b]).wait()
            pltpu.async_copy(rows[b], acc.at[dst_v.at[j]], ssem[b], add=True)
        return carry

    lax.fori_loop(0, NB // K, body, 0)
    for b in range(K):
        pltpu.make_async_copy(rows[b], acc.at[dst_v.at[0]], ssem[b]).wait()
    plsc.subcore_barrier()
    pltpu.sync_copy(acc.at[pl.ds(r0, RPT)], out_hbm.at[c, pl.ds(r0, RPT)])


def _norm_col(deg_ref):
    d = deg_ref[0] + deg_ref[1]
    return lax.rsqrt(jnp.maximum(d[:, 0:1], 1.0))


def _mm1_body(dego_ref, x_ref, w_ref, ha_ref, hb_ref):
    ns = _norm_col(dego_ref)
    h = jnp.dot(x_ref[...], w_ref[...], preferred_element_type=jnp.float32,
                precision=lax.Precision.HIGHEST)
    h = h * ns
    ha_ref[...] = h[:, :DC]
    hb_ref[...] = h[:, DC:]


def _mm2_body(pa_ref, pb_ref, dego_ref, degi_ref, w_ref, b_ref, h_ref):
    ns = _norm_col(dego_ref)
    nd = _norm_col(degi_ref)
    ta = jnp.maximum((pa_ref[0] + pa_ref[1]) * nd + b_ref[:, :DC], 0.0)
    tb = jnp.maximum((pb_ref[0] + pb_ref[1]) * nd + b_ref[:, DC:], 0.0)
    h = (jnp.dot(ta, w_ref[:DC], preferred_element_type=jnp.float32,
                 precision=lax.Precision.HIGHEST)
         + jnp.dot(tb, w_ref[DC:], preferred_element_type=jnp.float32,
                   precision=lax.Precision.HIGHEST))
    h_ref[...] = h * ns


def _fin_body(aggp_ref, degi_ref, b_ref, o_ref):
    nd = _norm_col(degi_ref)
    o_ref[...] = (aggp_ref[0] + aggp_ref[1]) * nd + b_ref[...]


_BLK = 640
_GRID = N1 // _BLK


def _deg_spec():
    return pl.BlockSpec((2, _BLK, 16), lambda i: (0, i, 0))


def _agg_spec():
    return pl.BlockSpec((2, _BLK, DC), lambda i: (0, i, 0))


def _tc_mm1(deg_out_p, x_pad, W1):
    return pl.pallas_call(
        _mm1_body,
        grid=(_GRID,),
        in_specs=[
            _deg_spec(),
            pl.BlockSpec((_BLK, D_IN), lambda i: (i, 0)),
            pl.BlockSpec((D_IN, D_H), lambda i: (0, 0)),
        ],
        out_specs=[
            pl.BlockSpec((_BLK, DC), lambda i: (i, 0)),
            pl.BlockSpec((_BLK, DC), lambda i: (i, 0)),
        ],
        out_shape=[
            jax.ShapeDtypeStruct((N1, DC), jnp.float32),
            jax.ShapeDtypeStruct((N1, DC), jnp.float32),
        ],
    )(deg_out_p, x_pad, W1)


def _tc_mm2(agg1a_p, agg1b_p, deg_out_p, deg_in_p, W2, b1):
    return pl.pallas_call(
        _mm2_body,
        grid=(_GRID,),
        in_specs=[
            _agg_spec(),
            _agg_spec(),
            _deg_spec(),
            _deg_spec(),
            pl.BlockSpec((D_H, D_OUT), lambda i: (0, 0)),
            pl.BlockSpec((1, D_H), lambda i: (0, 0)),
        ],
        out_specs=pl.BlockSpec((_BLK, D_OUT), lambda i: (i, 0)),
        out_shape=jax.ShapeDtypeStruct((N1, D_OUT), jnp.float32),
    )(agg1a_p, agg1b_p, deg_out_p, deg_in_p, W2, b1)


def _tc_fin(agg2_p, deg_in_p, b2):
    return pl.pallas_call(
        _fin_body,
        grid=(_GRID,),
        in_specs=[
            _agg_spec(),
            _deg_spec(),
            pl.BlockSpec((1, D_OUT), lambda i: (0, 0)),
        ],
        out_specs=pl.BlockSpec((_BLK, D_OUT), lambda i: (i, 0)),
        out_shape=jax.ShapeDtypeStruct((N1, D_OUT), jnp.float32),
    )(agg2_p, deg_in_p, b2)


def kernel(x, edge_index, W1, b1, W2, b2):
    src = edge_index[0]
    dst = edge_index[1]
    pad = jnp.full((EPAD - E,), N, dtype=jnp.int32)
    src_p = jnp.concatenate([src, pad]).reshape(NBT, B)
    dst_p = jnp.concatenate([dst, pad]).reshape(NBT, B)
    x_pad = jnp.zeros((N1, D_IN), jnp.float32).at[:N].set(x)

    ones16 = jnp.ones((B, 16), jnp.float32)
    z16 = jnp.zeros((RPT, 16), jnp.float32)
    z64 = jnp.zeros((RPT, DC), jnp.float32)

    deg_out_p, deg_in_p = _deg_kernel(src_p, dst_p, ones16, z16)
    h1a, h1b = _tc_mm1(deg_out_p, x_pad, W1)
    agg1a_p = _agg_kernel(h1a, src_p, dst_p, z64)
    agg1b_p = _agg_kernel(h1b, src_p, dst_p, z64)
    h2 = _tc_mm2(agg1a_p, agg1b_p, deg_out_p, deg_in_p, W2,
                 b1.reshape(1, D_H))
    agg2_p = _agg_kernel(h2, src_p, dst_p, z64)
    out = _tc_fin(agg2_p, deg_in_p, b2.reshape(1, D_OUT))
    return out[:N]


# speedup vs baseline: 1.8585x; 1.0658x over previous
import functools

import jax
import jax.numpy as jnp
from jax import lax
from jax.experimental import pallas as pl
from jax.experimental.pallas import tpu as pltpu
from jax.experimental.pallas import tpu_sc as plsc

N = 10000
E = 320000
D_IN = 128
D_H = 128
D_OUT = 64
DC = 64

N1 = 10240
RPT = N1 // 16
NW = 32
B = 128
K = 3
NBT = 2592
EPAD = NBT * B
FA, SA = 84, 78
FD, SD = 96, 66

_mesh = plsc.VectorSubcoreMesh(core_axis_name="c", subcore_axis_name="s")
_sc_params = pltpu.CompilerParams(use_tc_tiling_on_sc=False)


def _zero_buf(buf, width):
    zf = jnp.zeros((16,), jnp.float32)

    def zbody(i, carry):
        for q in range(width // 16):
            buf[i, pl.ds(16 * q, 16)] = zf
        return carry

    lax.fori_loop(0, B, zbody, 0)


def _zero_acc(acc, buf, r0):
    for t in range(RPT // B):
        pltpu.sync_copy(buf, acc.at[pl.ds(r0 + t * B, B)])


@functools.partial(
    pl.kernel,
    out_type=(
        jax.ShapeDtypeStruct((2, N1, 16), jnp.float32),
        jax.ShapeDtypeStruct((2, N1, 16), jnp.float32),
    ),
    mesh=_mesh,
    compiler_params=_sc_params,
    scratch_types=[
        pltpu.VMEM((FD, B), jnp.int32),
        pltpu.VMEM((FD, B), jnp.int32),
        pltpu.VMEM((B, 16), jnp.float32),
        pltpu.VMEM((B, 16), jnp.float32),
        pltpu.VMEM_SHARED((N1, 16), jnp.float32),
        pltpu.VMEM_SHARED((N1, 16), jnp.float32),
        pltpu.SemaphoreType.DMA,
        pltpu.SemaphoreType.DMA,
    ],
)
def _deg_kernel(src_hbm, dst_hbm, ones_hbm,
                deg_out_hbm, deg_in_hbm,
                src_v, dst_v, ones_v, zbuf, acc_o, acc_i, sem, isem):
    c = lax.axis_index("c")
    s = lax.axis_index("s")
    r0 = s * RPT
    pltpu.async_copy(ones_hbm, ones_v, isem)
    _zero_buf(zbuf, 16)
    _zero_acc(acc_o, zbuf, r0)
    _zero_acc(acc_i, zbuf, r0)
    pltpu.make_async_copy(ones_hbm, ones_v, isem).wait()

    def run(count, start):
        pltpu.sync_copy(src_hbm.at[pl.ds(start, count)],
                        src_v.at[pl.ds(0, count)])
        pltpu.sync_copy(dst_hbm.at[pl.ds(start, count)],
                        dst_v.at[pl.ds(0, count)])
        plsc.subcore_barrier()

        def body(g, carry):
            for q in range(K):
                j = K * g + q
                pltpu.async_copy(ones_v, acc_o.at[src_v.at[j]], sem,
                                 add=True)
                pltpu.async_copy(ones_v, acc_i.at[dst_v.at[j]], sem,
                                 add=True)
            for q in range(2 * K):
                pltpu.make_async_copy(ones_v, acc_i.at[dst_v.at[0]],
                                      sem).wait()
            return carry

        lax.fori_loop(0, count // K, body, 0)

    @pl.when(c == 0)
    def _fast():
        run(FD, s * FD)

    @pl.when(c == 1)
    def _slow():
        run(SD, 16 * FD + s * SD)

    plsc.subcore_barrier()
    pltpu.sync_copy(acc_o.at[pl.ds(r0, RPT)], deg_out_hbm.at[c, pl.ds(r0, RPT)])
    pltpu.sync_copy(acc_i.at[pl.ds(r0, RPT)], deg_in_hbm.at[c, pl.ds(r0, RPT)])


def _agg_body(m_hbm, out_hbm, src_v, dst_v, rows, gsem, ssem, table, acc,
              c, s, count):
    r0 = s * RPT
    pltpu.async_copy(m_hbm.at[pl.ds(r0, RPT)], table.at[pl.ds(r0, RPT)],
                     gsem[0])
    _zero_buf(rows[0], DC)
    _zero_acc(acc, rows[0], r0)
    pltpu.make_async_copy(m_hbm.at[pl.ds(r0, RPT)], table.at[pl.ds(r0, RPT)],
                          gsem[0]).wait()
    plsc.subcore_barrier()

    def body(g, carry):
        for b in range(K):
            j = K * g + b

            @pl.when(g > 0)
            def _drain_scatter(b=b):
                pltpu.make_async_copy(rows[b], acc.at[dst_v.at[0]],
                                      ssem[b]).wait()

            pltpu.async_copy(table.at[src_v.at[j]], rows[b], gsem[b])
        for b in range(K):
            j = K * g + b
            pltpu.make_async_copy(table.at[src_v.at[0]], rows[b],
                                  gsem[b]).wait()
            pltpu.async_copy(rows[b], acc.at[dst_v.at[j]], ssem[b],
                             add=True)
        return carry

    lax.fori_loop(0, count // K, body, 0)
    for b in range(K):
        pltpu.make_async_copy(rows[b], acc.at[dst_v.at[0]], ssem[b]).wait()
    plsc.subcore_barrier()
    pltpu.sync_copy(acc.at[pl.ds(r0, RPT)], out_hbm.at[c, pl.ds(r0, RPT)])
    plsc.subcore_barrier()


_AGG_SCRATCH = [
    pltpu.VMEM((FA, B), jnp.int32),
    pltpu.VMEM((FA, B), jnp.int32),
    [pltpu.VMEM((B, DC), jnp.float32)] * K,
    [pltpu.SemaphoreType.DMA] * K,
    [pltpu.SemaphoreType.DMA] * K,
    pltpu.VMEM_SHARED((N1, DC), jnp.float32),
    pltpu.VMEM_SHARED((N1, DC), jnp.float32),
]


def _load_idx(src_hbm, dst_hbm, src_v, dst_v, count, start):
    pltpu.sync_copy(src_hbm.at[pl.ds(start, count)],
                    src_v.at[pl.ds(0, count)])
    pltpu.sync_copy(dst_hbm.at[pl.ds(start, count)],
                    dst_v.at[pl.ds(0, count)])


@functools.partial(
    pl.kernel,
    out_type=(
        jax.ShapeDtypeStruct((2, N1, DC), jnp.float32),
        jax.ShapeDtypeStruct((2, N1, DC), jnp.float32),
    ),
    mesh=_mesh,
    compiler_params=_sc_params,
    scratch_types=_AGG_SCRATCH,
)
def _agg2_kernel(ma_hbm, mb_hbm, src_hbm, dst_hbm, outa_hbm, outb_hbm,
                 src_v, dst_v, rows, gsem, ssem, table, acc):
    c = lax.axis_index("c")
    s = lax.axis_index("s")

    def run(count, start):
        _load_idx(src_hbm, dst_hbm, src_v, dst_v, count, start)
        _agg_body(ma_hbm, outa_hbm, src_v, dst_v, rows, gsem, ssem,
                  table, acc, c, s, count)
        _agg_body(mb_hbm, outb_hbm, src_v, dst_v, rows, gsem, ssem,
                  table, acc, c, s, count)

    @pl.when(c == 0)
    def _fast():
        run(FA, s * FA)

    @pl.when(c == 1)
    def _slow():
        run(SA, 16 * FA + s * SA)


@functools.partial(
    pl.kernel,
    out_type=jax.ShapeDtypeStruct((2, N1, DC), jnp.float32),
    mesh=_mesh,
    compiler_params=_sc_params,
    scratch_types=_AGG_SCRATCH,
)
def _agg1_kernel(m_hbm, src_hbm, dst_hbm, out_hbm,
                 src_v, dst_v, rows, gsem, ssem, table, acc):
    c = lax.axis_index("c")
    s = lax.axis_index("s")

    def run(count, start):
        _load_idx(src_hbm, dst_hbm, src_v, dst_v, count, start)
        _agg_body(m_hbm, out_hbm, src_v, dst_v, rows, gsem, ssem,
                  table, acc, c, s, count)

    @pl.when(c == 0)
    def _fast():
        run(FA, s * FA)

    @pl.when(c == 1)
    def _slow():
        run(SA, 16 * FA + s * SA)


def _norm_col(deg_ref):
    d = deg_ref[0] + deg_ref[1]
    return lax.rsqrt(jnp.maximum(d[:, 0:1], 1.0))


def _mm1_body(dego_ref, x_ref, w_ref, ha_ref, hb_ref):
    ns = _norm_col(dego_ref)
    h = jnp.dot(x_ref[...], w_ref[...], preferred_element_type=jnp.float32,
                precision=lax.Precision.HIGHEST)
    h = h * ns
    ha_ref[...] = h[:, :DC]
    hb_ref[...] = h[:, DC:]


def _mm2_body(pa_ref, pb_ref, dego_ref, degi_ref, w_ref, b_ref, h_ref):
    ns = _norm_col(dego_ref)
    nd = _norm_col(degi_ref)
    ta = jnp.maximum((pa_ref[0] + pa_ref[1]) * nd + b_ref[:, :DC], 0.0)
    tb = jnp.maximum((pb_ref[0] + pb_ref[1]) * nd + b_ref[:, DC:], 0.0)
    h = (jnp.dot(ta, w_ref[:DC], preferred_element_type=jnp.float32,
                 precision=lax.Precision.HIGHEST)
         + jnp.dot(tb, w_ref[DC:], preferred_element_type=jnp.float32,
                   precision=lax.Precision.HIGHEST))
    h_ref[...] = h * ns


def _fin_body(aggp_ref, degi_ref, b_ref, o_ref):
    nd = _norm_col(degi_ref)
    o_ref[...] = (aggp_ref[0] + aggp_ref[1]) * nd + b_ref[...]


_BLK = 640
_GRID = N1 // _BLK


def _deg_spec():
    return pl.BlockSpec((2, _BLK, 16), lambda i: (0, i, 0))


def _agg_spec():
    return pl.BlockSpec((2, _BLK, DC), lambda i: (0, i, 0))


def _tc_mm1(deg_out_p, x_pad, W1):
    return pl.pallas_call(
        _mm1_body,
        grid=(_GRID,),
        in_specs=[
            _deg_spec(),
            pl.BlockSpec((_BLK, D_IN), lambda i: (i, 0)),
            pl.BlockSpec((D_IN, D_H), lambda i: (0, 0)),
        ],
        out_specs=[
            pl.BlockSpec((_BLK, DC), lambda i: (i, 0)),
            pl.BlockSpec((_BLK, DC), lambda i: (i, 0)),
        ],
        out_shape=[
            jax.ShapeDtypeStruct((N1, DC), jnp.float32),
            jax.ShapeDtypeStruct((N1, DC), jnp.float32),
        ],
    )(deg_out_p, x_pad, W1)


def _tc_mm2(agg1a_p, agg1b_p, deg_out_p, deg_in_p, W2, b1):
    return pl.pallas_call(
        _mm2_body,
        grid=(_GRID,),
        in_specs=[
            _agg_spec(),
            _agg_spec(),
            _deg_spec(),
            _deg_spec(),
            pl.BlockSpec((D_H, D_OUT), lambda i: (0, 0)),
            pl.BlockSpec((1, D_H), lambda i: (0, 0)),
        ],
        out_specs=pl.BlockSpec((_BLK, D_OUT), lambda i: (i, 0)),
        out_shape=jax.ShapeDtypeStruct((N1, D_OUT), jnp.float32),
    )(agg1a_p, agg1b_p, deg_out_p, deg_in_p, W2, b1)


def _tc_fin(agg2_p, deg_in_p, b2):
    return pl.pallas_call(
        _fin_body,
        grid=(_GRID,),
        in_specs=[
            _agg_spec(),
            _deg_spec(),
            pl.BlockSpec((1, D_OUT), lambda i: (0, 0)),
        ],
        out_specs=pl.BlockSpec((_BLK, D_OUT), lambda i: (i, 0)),
        out_shape=jax.ShapeDtypeStruct((N1, D_OUT), jnp.float32),
    )(agg2_p, deg_in_p, b2)


def kernel(x, edge_index, W1, b1, W2, b2):
    src = edge_index[0]
    dst = edge_index[1]
    pad = jnp.full((EPAD - E,), N, dtype=jnp.int32)
    src_p = jnp.concatenate([src, pad]).reshape(NBT, B)
    dst_p = jnp.concatenate([dst, pad]).reshape(NBT, B)
    x_pad = jnp.zeros((N1, D_IN), jnp.float32).at[:N].set(x)

    ones16 = jnp.ones((B, 16), jnp.float32)

    deg_out_p, deg_in_p = _deg_kernel(src_p, dst_p, ones16)
    h1a, h1b = _tc_mm1(deg_out_p, x_pad, W1)
    agg1a_p, agg1b_p = _agg2_kernel(h1a, h1b, src_p, dst_p)
    h2 = _tc_mm2(agg1a_p, agg1b_p, deg_out_p, deg_in_p, W2,
                 b1.reshape(1, D_H))
    agg2_p = _agg1_kernel(h2, src_p, dst_p)
    out = _tc_fin(agg2_p, deg_in_p, b2.reshape(1, D_OUT))
    return out[:N]


# speedup vs baseline: 1.9311x; 1.0391x over previous
import functools

import jax
import jax.numpy as jnp
from jax import lax
from jax.experimental import pallas as pl
from jax.experimental.pallas import tpu as pltpu
from jax.experimental.pallas import tpu_sc as plsc

N = 10000
E = 320000
D_IN = 128
D_H = 128
D_OUT = 64
DC = 64

N1 = 10240
RPT = N1 // 16
NW = 32
B = 128
K = 3
NBT = 2592
EPAD = NBT * B
FA, SA = 84, 78
FD, SD = 96, 66

_mesh = plsc.VectorSubcoreMesh(core_axis_name="c", subcore_axis_name="s")
_sc_params = pltpu.CompilerParams(use_tc_tiling_on_sc=False)


def _zero_buf(buf, width):
    zf = jnp.zeros((16,), jnp.float32)

    def zbody(i, carry):
        for q in range(width // 16):
            buf[i, pl.ds(16 * q, 16)] = zf
        return carry

    lax.fori_loop(0, B, zbody, 0)


def _zero_acc(acc, buf, r0):
    for t in range(RPT // B):
        pltpu.sync_copy(buf, acc.at[pl.ds(r0 + t * B, B)])


NBD = NBT // 16


@functools.partial(
    pl.kernel,
    out_type=(
        jax.ShapeDtypeStruct((N1, 16), jnp.float32),
        jax.ShapeDtypeStruct((N1, 16), jnp.float32),
    ),
    mesh=_mesh,
    compiler_params=_sc_params,
    scratch_types=[
        pltpu.VMEM((NBD, B), jnp.int32),
        pltpu.VMEM((B, 16), jnp.float32),
        pltpu.VMEM((B, 16), jnp.float32),
        pltpu.VMEM_SHARED((N1, 16), jnp.float32),
        pltpu.SemaphoreType.DMA,
        pltpu.SemaphoreType.DMA,
    ],
)
def _deg_kernel(src_hbm, dst_hbm, ones_hbm,
                deg_out_hbm, deg_in_hbm,
                idx_v, ones_v, zbuf, acc, sem, isem):
    c = lax.axis_index("c")
    s = lax.axis_index("s")
    r0 = s * RPT
    pltpu.async_copy(ones_hbm, ones_v, isem)
    _zero_buf(zbuf, 16)
    _zero_acc(acc, zbuf, r0)
    pltpu.make_async_copy(ones_hbm, ones_v, isem).wait()

    def run(idx_hbm, out_hbm):
        pltpu.sync_copy(idx_hbm.at[pl.ds(s * NBD, NBD)], idx_v)
        plsc.subcore_barrier()

        def body(g, carry):
            for q in range(2 * K):
                j = 2 * K * g + q
                pltpu.async_copy(ones_v, acc.at[idx_v.at[j]], sem, add=True)
            for q in range(2 * K):
                pltpu.make_async_copy(ones_v, acc.at[idx_v.at[0]],
                                      sem).wait()
            return carry

        lax.fori_loop(0, NBD // (2 * K), body, 0)
        plsc.subcore_barrier()
        pltpu.sync_copy(acc.at[pl.ds(r0, RPT)], out_hbm.at[pl.ds(r0, RPT)])

    @pl.when(c == 0)
    def _src_side():
        run(src_hbm, deg_out_hbm)

    @pl.when(c == 1)
    def _dst_side():
        run(dst_hbm, deg_in_hbm)


def _agg_body(m_hbm, out_hbm, src_v, dst_v, rows, gsem, ssem, table, acc,
              c, s, count):
    r0 = s * RPT
    pltpu.async_copy(m_hbm.at[pl.ds(r0, RPT)], table.at[pl.ds(r0, RPT)],
                     gsem[0])
    _zero_buf(rows[0], DC)
    _zero_acc(acc, rows[0], r0)
    pltpu.make_async_copy(m_hbm.at[pl.ds(r0, RPT)], table.at[pl.ds(r0, RPT)],
                          gsem[0]).wait()
    plsc.subcore_barrier()

    def body(g, carry):
        for b in range(K):
            j = K * g + b

            @pl.when(g > 0)
            def _drain_scatter(b=b):
                pltpu.make_async_copy(rows[b], acc.at[dst_v.at[0]],
                                      ssem[b]).wait()

            pltpu.async_copy(table.at[src_v.at[j]], rows[b], gsem[b])
        for b in range(K):
            j = K * g + b
            pltpu.make_async_copy(table.at[src_v.at[0]], rows[b],
                                  gsem[b]).wait()
            pltpu.async_copy(rows[b], acc.at[dst_v.at[j]], ssem[b],
                             add=True)
        return carry

    lax.fori_loop(0, count // K, body, 0)
    for b in range(K):
        pltpu.make_async_copy(rows[b], acc.at[dst_v.at[0]], ssem[b]).wait()
    plsc.subcore_barrier()
    pltpu.sync_copy(acc.at[pl.ds(r0, RPT)], out_hbm.at[c, pl.ds(r0, RPT)])
    plsc.subcore_barrier()


_AGG_SCRATCH = [
    pltpu.VMEM((FA, B), jnp.int32),
    pltpu.VMEM((FA, B), jnp.int32),
    [pltpu.VMEM((B, DC), jnp.float32)] * K,
    [pltpu.SemaphoreType.DMA] * K,
    [pltpu.SemaphoreType.DMA] * K,
    pltpu.VMEM_SHARED((N1, DC), jnp.float32),
    pltpu.VMEM_SHARED((N1, DC), jnp.float32),
]


def _load_idx(src_hbm, dst_hbm, src_v, dst_v, count, start):
    pltpu.sync_copy(src_hbm.at[pl.ds(start, count)],
                    src_v.at[pl.ds(0, count)])
    pltpu.sync_copy(dst_hbm.at[pl.ds(start, count)],
                    dst_v.at[pl.ds(0, count)])


@functools.partial(
    pl.kernel,
    out_type=(
        jax.ShapeDtypeStruct((2, N1, DC), jnp.float32),
        jax.ShapeDtypeStruct((2, N1, DC), jnp.float32),
    ),
    mesh=_mesh,
    compiler_params=_sc_params,
    scratch_types=_AGG_SCRATCH,
)
def _agg2_kernel(ma_hbm, mb_hbm, src_hbm, dst_hbm, outa_hbm, outb_hbm,
                 src_v, dst_v, rows, gsem, ssem, table, acc):
    c = lax.axis_index("c")
    s = lax.axis_index("s")

    def run(count, start):
        _load_idx(src_hbm, dst_hbm, src_v, dst_v, count, start)
        _agg_body(ma_hbm, outa_hbm, src_v, dst_v, rows, gsem, ssem,
                  table, acc, c, s, count)
        _agg_body(mb_hbm, outb_hbm, src_v, dst_v, rows, gsem, ssem,
                  table, acc, c, s, count)

    @pl.when(c == 0)
    def _fast():
        run(FA, s * FA)

    @pl.when(c == 1)
    def _slow():
        run(SA, 16 * FA + s * SA)


@functools.partial(
    pl.kernel,
    out_type=jax.ShapeDtypeStruct((2, N1, DC), jnp.float32),
    mesh=_mesh,
    compiler_params=_sc_params,
    scratch_types=_AGG_SCRATCH,
)
def _agg1_kernel(m_hbm, src_hbm, dst_hbm, out_hbm,
                 src_v, dst_v, rows, gsem, ssem, table, acc):
    c = lax.axis_index("c")
    s = lax.axis_index("s")

    def run(count, start):
        _load_idx(src_hbm, dst_hbm, src_v, dst_v, count, start)
        _agg_body(m_hbm, out_hbm, src_v, dst_v, rows, gsem, ssem,
                  table, acc, c, s, count)

    @pl.when(c == 0)
    def _fast():
        run(FA, s * FA)

    @pl.when(c == 1)
    def _slow():
        run(SA, 16 * FA + s * SA)


def _norm_col(deg_ref):
    d = deg_ref[...]
    return lax.rsqrt(jnp.maximum(d[:, 0:1], 1.0))


def _mm1_body(dego_ref, x_ref, w_ref, ha_ref, hb_ref):
    ns = _norm_col(dego_ref)
    h = jnp.dot(x_ref[...], w_ref[...], preferred_element_type=jnp.float32,
                precision=lax.Precision.HIGHEST)
    h = h * ns
    ha_ref[...] = h[:, :DC]
    hb_ref[...] = h[:, DC:]


def _mm2_body(pa_ref, pb_ref, dego_ref, degi_ref, w_ref, b_ref, h_ref):
    ns = _norm_col(dego_ref)
    nd = _norm_col(degi_ref)
    ta = jnp.maximum((pa_ref[0] + pa_ref[1]) * nd + b_ref[:, :DC], 0.0)
    tb = jnp.maximum((pb_ref[0] + pb_ref[1]) * nd + b_ref[:, DC:], 0.0)
    h = (jnp.dot(ta, w_ref[:DC], preferred_element_type=jnp.float32,
                 precision=lax.Precision.HIGHEST)
         + jnp.dot(tb, w_ref[DC:], preferred_element_type=jnp.float32,
                   precision=lax.Precision.HIGHEST))
    h_ref[...] = h * ns


def _fin_body(aggp_ref, degi_ref, b_ref, o_ref):
    nd = _norm_col(degi_ref)
    o_ref[...] = (aggp_ref[0] + aggp_ref[1]) * nd + b_ref[...]


_BLK = 640
_GRID = N1 // _BLK


def _deg_spec():
    return pl.BlockSpec((_BLK, 16), lambda i: (i, 0))


def _agg_spec():
    return pl.BlockSpec((2, _BLK, DC), lambda i: (0, i, 0))


def _tc_mm1(deg_out, x, W1):
    return pl.pallas_call(
        _mm1_body,
        grid=(_GRID,),
        in_specs=[
            _deg_spec(),
            pl.BlockSpec((_BLK, D_IN), lambda i: (i, 0)),
            pl.BlockSpec((D_IN, D_H), lambda i: (0, 0)),
        ],
        out_specs=[
            pl.BlockSpec((_BLK, DC), lambda i: (i, 0)),
            pl.BlockSpec((_BLK, DC), lambda i: (i, 0)),
        ],
        out_shape=[
            jax.ShapeDtypeStruct((N1, DC), jnp.float32),
            jax.ShapeDtypeStruct((N1, DC), jnp.float32),
        ],
    )(deg_out, x, W1)


def _tc_mm2(agg1a_p, agg1b_p, deg_out, deg_in, W2, b1):
    return pl.pallas_call(
        _mm2_body,
        grid=(_GRID,),
        in_specs=[
            _agg_spec(),
            _agg_spec(),
            _deg_spec(),
            _deg_spec(),
            pl.BlockSpec((D_H, D_OUT), lambda i: (0, 0)),
            pl.BlockSpec((1, D_H), lambda i: (0, 0)),
        ],
        out_specs=pl.BlockSpec((_BLK, D_OUT), lambda i: (i, 0)),
        out_shape=jax.ShapeDtypeStruct((N1, D_OUT), jnp.float32),
    )(agg1a_p, agg1b_p, deg_out, deg_in, W2, b1)


def _tc_fin(agg2_p, deg_in, b2):
    return pl.pallas_call(
        _fin_body,
        grid=(_GRID,),
        in_specs=[
            _agg_spec(),
            _deg_spec(),
            pl.BlockSpec((1, D_OUT), lambda i: (0, 0)),
        ],
        out_specs=pl.BlockSpec((_BLK, D_OUT), lambda i: (i, 0)),
        out_shape=jax.ShapeDtypeStruct((N, D_OUT), jnp.float32),
    )(agg2_p, deg_in, b2)


def kernel(x, edge_index, W1, b1, W2, b2):
    src = edge_index[0]
    dst = edge_index[1]
    pad = jnp.full((EPAD - E,), N, dtype=jnp.int32)
    src_p = jnp.concatenate([src, pad]).reshape(NBT, B)
    dst_p = jnp.concatenate([dst, pad]).reshape(NBT, B)

    ones16 = jnp.ones((B, 16), jnp.float32)

    deg_out, deg_in = _deg_kernel(src_p, dst_p, ones16)
    h1a, h1b = _tc_mm1(deg_out, x, W1)
    agg1a_p, agg1b_p = _agg2_kernel(h1a, h1b, src_p, dst_p)
    h2 = _tc_mm2(agg1a_p, agg1b_p, deg_out, deg_in, W2,
                 b1.reshape(1, D_H))
    agg2_p = _agg1_kernel(h2, src_p, dst_p)
    return _tc_fin(agg2_p, deg_in, b2.reshape(1, D_OUT))


# speedup vs baseline: 1.9605x; 1.0152x over previous
import functools

import jax
import jax.numpy as jnp
from jax import lax
from jax.experimental import pallas as pl
from jax.experimental.pallas import tpu as pltpu
from jax.experimental.pallas import tpu_sc as plsc

N = 10000
E = 320000
D_IN = 128
D_H = 128
D_OUT = 64
DC = 64

N1 = 10240
RPT = N1 // 16
NW = 32
B = 128
K = 3
NBT = 2592
EPAD = NBT * B
FA, SA = 84, 78
FD, SD = 96, 66

_mesh = plsc.VectorSubcoreMesh(core_axis_name="c", subcore_axis_name="s")
_sc_params = pltpu.CompilerParams(use_tc_tiling_on_sc=False)


def _zero_buf(buf, width):
    zf = jnp.zeros((16,), jnp.float32)

    def zbody(i, carry):
        for q in range(width // 16):
            buf[i, pl.ds(16 * q, 16)] = zf
        return carry

    lax.fori_loop(0, B, zbody, 0)


def _zero_acc(acc, buf, r0):
    for t in range(RPT // B):
        pltpu.sync_copy(buf, acc.at[pl.ds(r0 + t * B, B)])


NBD = NBT // 16


@functools.partial(
    pl.kernel,
    out_type=(
        jax.ShapeDtypeStruct((N1, 8), jnp.float32),
        jax.ShapeDtypeStruct((N1, 8), jnp.float32),
    ),
    mesh=_mesh,
    compiler_params=_sc_params,
    scratch_types=[
        pltpu.VMEM((NBD, B), jnp.int32),
        pltpu.VMEM((B, 8), jnp.float32),
        pltpu.VMEM((B, 8), jnp.float32),
        pltpu.VMEM_SHARED((N1, 8), jnp.float32),
        pltpu.SemaphoreType.DMA,
        pltpu.SemaphoreType.DMA,
    ],
)
def _deg_kernel(src_hbm, dst_hbm, ones_hbm,
                deg_out_hbm, deg_in_hbm,
                idx_v, ones_v, zbuf, acc, sem, isem):
    c = lax.axis_index("c")
    s = lax.axis_index("s")
    r0 = s * RPT
    pltpu.async_copy(ones_hbm, ones_v, isem)
    _zero_buf(zbuf, 8)
    _zero_acc(acc, zbuf, r0)
    pltpu.make_async_copy(ones_hbm, ones_v, isem).wait()

    def run(idx_hbm, out_hbm):
        pltpu.sync_copy(idx_hbm.at[pl.ds(s * NBD, NBD)], idx_v)
        plsc.subcore_barrier()

        def body(g, carry):
            for q in range(2 * K):
                j = 2 * K * g + q
                pltpu.async_copy(ones_v, acc.at[idx_v.at[j]], sem, add=True)
            for q in range(2 * K):
                pltpu.make_async_copy(ones_v, acc.at[idx_v.at[0]],
                                      sem).wait()
            return carry

        lax.fori_loop(0, NBD // (2 * K), body, 0)
        plsc.subcore_barrier()
        pltpu.sync_copy(acc.at[pl.ds(r0, RPT)], out_hbm.at[pl.ds(r0, RPT)])

    @pl.when(c == 0)
    def _src_side():
        run(src_hbm, deg_out_hbm)

    @pl.when(c == 1)
    def _dst_side():
        run(dst_hbm, deg_in_hbm)


def _agg_body(m_hbm, out_hbm, src_v, dst_v, rows, gsem, ssem, table, acc,
              c, s, count):
    r0 = s * RPT
    pltpu.async_copy(m_hbm.at[pl.ds(r0, RPT)], table.at[pl.ds(r0, RPT)],
                     gsem[0])
    _zero_buf(rows[0], DC)
    _zero_acc(acc, rows[0], r0)
    pltpu.make_async_copy(m_hbm.at[pl.ds(r0, RPT)], table.at[pl.ds(r0, RPT)],
                          gsem[0]).wait()
    plsc.subcore_barrier()

    def body(g, carry):
        for b in range(K):
            j = K * g + b

            @pl.when(g > 0)
            def _drain_scatter(b=b):
                pltpu.make_async_copy(rows[b], acc.at[dst_v.at[0]],
                                      ssem[b]).wait()

            pltpu.async_copy(table.at[src_v.at[j]], rows[b], gsem[b])
        for b in range(K):
            j = K * g + b
            pltpu.make_async_copy(table.at[src_v.at[0]], rows[b],
                                  gsem[b]).wait()
            pltpu.async_copy(rows[b], acc.at[dst_v.at[j]], ssem[b],
                             add=True)
        return carry

    lax.fori_loop(0, count // K, body, 0)
    for b in range(K):
        pltpu.make_async_copy(rows[b], acc.at[dst_v.at[0]], ssem[b]).wait()
    plsc.subcore_barrier()
    pltpu.sync_copy(acc.at[pl.ds(r0, RPT)], out_hbm.at[c, pl.ds(r0, RPT)])
    plsc.subcore_barrier()


_AGG_SCRATCH = [
    pltpu.VMEM((FA, B), jnp.int32),
    pltpu.VMEM((FA, B), jnp.int32),
    [pltpu.VMEM((B, DC), jnp.float32)] * K,
    [pltpu.SemaphoreType.DMA] * K,
    [pltpu.SemaphoreType.DMA] * K,
    pltpu.VMEM_SHARED((N1, DC), jnp.float32),
    pltpu.VMEM_SHARED((N1, DC), jnp.float32),
]


def _load_idx(src_hbm, dst_hbm, src_v, dst_v, count, start):
    pltpu.sync_copy(src_hbm.at[pl.ds(start, count)],
                    src_v.at[pl.ds(0, count)])
    pltpu.sync_copy(dst_hbm.at[pl.ds(start, count)],
                    dst_v.at[pl.ds(0, count)])


@functools.partial(
    pl.kernel,
    out_type=(
        jax.ShapeDtypeStruct((2, N1, DC), jnp.float32),
        jax.ShapeDtypeStruct((2, N1, DC), jnp.float32),
    ),
    mesh=_mesh,
    compiler_params=_sc_params,
    scratch_types=_AGG_SCRATCH,
)
def _agg2_kernel(ma_hbm, mb_hbm, src_hbm, dst_hbm, outa_hbm, outb_hbm,
                 src_v, dst_v, rows, gsem, ssem, table, acc):
    c = lax.axis_index("c")
    s = lax.axis_index("s")

    def run(count, start):
        _load_idx(src_hbm, dst_hbm, src_v, dst_v, count, start)
        _agg_body(ma_hbm, outa_hbm, src_v, dst_v, rows, gsem, ssem,
                  table, acc, c, s, count)
        _agg_body(mb_hbm, outb_hbm, src_v, dst_v, rows, gsem, ssem,
                  table, acc, c, s, count)

    @pl.when(c == 0)
    def _fast():
        run(FA, s * FA)

    @pl.when(c == 1)
    def _slow():
        run(SA, 16 * FA + s * SA)


@functools.partial(
    pl.kernel,
    out_type=jax.ShapeDtypeStruct((2, N1, DC), jnp.float32),
    mesh=_mesh,
    compiler_params=_sc_params,
    scratch_types=_AGG_SCRATCH,
)
def _agg1_kernel(m_hbm, src_hbm, dst_hbm, out_hbm,
                 src_v, dst_v, rows, gsem, ssem, table, acc):
    c = lax.axis_index("c")
    s = lax.axis_index("s")

    def run(count, start):
        _load_idx(src_hbm, dst_hbm, src_v, dst_v, count, start)
        _agg_body(m_hbm, out_hbm, src_v, dst_v, rows, gsem, ssem,
                  table, acc, c, s, count)

    @pl.when(c == 0)
    def _fast():
        run(FA, s * FA)

    @pl.when(c == 1)
    def _slow():
        run(SA, 16 * FA + s * SA)


def _norm_col(deg_ref):
    d = deg_ref[...]
    return lax.rsqrt(jnp.maximum(d[:, 0:1], 1.0))


def _mm1_body(dego_ref, x_ref, w_ref, ha_ref, hb_ref):
    ns = _norm_col(dego_ref)
    h = jnp.dot(x_ref[...], w_ref[...], preferred_element_type=jnp.float32,
                precision=lax.Precision.HIGHEST)
    h = h * ns
    ha_ref[...] = h[:, :DC]
    hb_ref[...] = h[:, DC:]


def _mm2_body(pa_ref, pb_ref, dego_ref, degi_ref, w_ref, b_ref, h_ref):
    ns = _norm_col(dego_ref)
    nd = _norm_col(degi_ref)
    ta = jnp.maximum((pa_ref[0] + pa_ref[1]) * nd + b_ref[:, :DC], 0.0)
    tb = jnp.maximum((pb_ref[0] + pb_ref[1]) * nd + b_ref[:, DC:], 0.0)
    h = (jnp.dot(ta, w_ref[:DC], preferred_element_type=jnp.float32,
                 precision=lax.Precision.HIGHEST)
         + jnp.dot(tb, w_ref[DC:], preferred_element_type=jnp.float32,
                   precision=lax.Precision.HIGHEST))
    h_ref[...] = h * ns


def _fin_body(aggp_ref, degi_ref, b_ref, o_ref):
    nd = _norm_col(degi_ref)
    o_ref[...] = (aggp_ref[0] + aggp_ref[1]) * nd + b_ref[...]


_BLK = 640
_GRID = N1 // _BLK


def _deg_spec():
    return pl.BlockSpec((_BLK, 8), lambda i: (i, 0))


def _agg_spec():
    return pl.BlockSpec((2, _BLK, DC), lambda i: (0, i, 0))


def _tc_mm1(deg_out, x, W1):
    return pl.pallas_call(
        _mm1_body,
        grid=(_GRID,),
        in_specs=[
            _deg_spec(),
            pl.BlockSpec((_BLK, D_IN), lambda i: (i, 0)),
            pl.BlockSpec((D_IN, D_H), lambda i: (0, 0)),
        ],
        out_specs=[
            pl.BlockSpec((_BLK, DC), lambda i: (i, 0)),
            pl.BlockSpec((_BLK, DC), lambda i: (i, 0)),
        ],
        out_shape=[
            jax.ShapeDtypeStruct((N1, DC), jnp.float32),
            jax.ShapeDtypeStruct((N1, DC), jnp.float32),
        ],
    )(deg_out, x, W1)


def _tc_mm2(agg1a_p, agg1b_p, deg_out, deg_in, W2, b1):
    return pl.pallas_call(
        _mm2_body,
        grid=(_GRID,),
        in_specs=[
            _agg_spec(),
            _agg_spec(),
            _deg_spec(),
            _deg_spec(),
            pl.BlockSpec((D_H, D_OUT), lambda i: (0, 0)),
            pl.BlockSpec((1, D_H), lambda i: (0, 0)),
        ],
        out_specs=pl.BlockSpec((_BLK, D_OUT), lambda i: (i, 0)),
        out_shape=jax.ShapeDtypeStruct((N1, D_OUT), jnp.float32),
    )(agg1a_p, agg1b_p, deg_out, deg_in, W2, b1)


def _tc_fin(agg2_p, deg_in, b2):
    return pl.pallas_call(
        _fin_body,
        grid=(_GRID,),
        in_specs=[
            _agg_spec(),
            _deg_spec(),
            pl.BlockSpec((1, D_OUT), lambda i: (0, 0)),
        ],
        out_specs=pl.BlockSpec((_BLK, D_OUT), lambda i: (i, 0)),
        out_shape=jax.ShapeDtypeStruct((N, D_OUT), jnp.float32),
    )(agg2_p, deg_in, b2)


def kernel(x, edge_index, W1, b1, W2, b2):
    src = edge_index[0]
    dst = edge_index[1]
    pad = jnp.full((EPAD - E,), N, dtype=jnp.int32)
    src_p = jnp.concatenate([src, pad]).reshape(NBT, B)
    dst_p = jnp.concatenate([dst, pad]).reshape(NBT, B)

    ones16 = jnp.ones((B, 8), jnp.float32)

    deg_out, deg_in = _deg_kernel(src_p, dst_p, ones16)
    h1a, h1b = _tc_mm1(deg_out, x, W1)
    agg1a_p, agg1b_p = _agg2_kernel(h1a, h1b, src_p, dst_p)
    h2 = _tc_mm2(agg1a_p, agg1b_p, deg_out, deg_in, W2,
                 b1.reshape(1, D_H))
    agg2_p = _agg1_kernel(h2, src_p, dst_p)
    return _tc_fin(agg2_p, deg_in, b2.reshape(1, D_OUT))
